# Initial kernel scaffold; baseline (speedup 1.0000x reference)
import jax
import jax.numpy as jnp
from jax.experimental import pallas as pl


def kernel(z, pos, batch, ptr, edge_index, emb_table, emb_W, emb_b, freq, msg_in_W, msg_in_b, rbf_W, rbf_b, msg_out_W, msg_out_b, fc_W, fc_b):
    raise NotImplementedError("write your pallas kernel here")


# speedup vs baseline: 2.0380x; 2.0380x over previous
import functools

import jax
import jax.numpy as jnp
from jax import lax
from jax.experimental import pallas as pl
from jax.experimental.pallas import tpu as pltpu
from jax.experimental.pallas import tpu_sc as plsc

_N = 10000
_E = 320000
_D = 128
_R = 32
_NB = 4
_HID = 5
_NTYPES = 101
_CUTOFF = 5.0

_NP = 10240
_NC = 2
_NS = 16
_NW = _NC * _NS
_EPW = _E // _NW
_CH = 80
_CPW = _EPW // _CH
_NCHUNK = _E // _CH
_RPS = _NP // _NS

_TN = 256
_TE = 512


def _mm(a, b):
    return lax.dot_general(a, b, (((a.ndim - 1,), (0,)), ((), ())),
                           precision=lax.Precision.HIGHEST,
                           preferred_element_type=jnp.float32)


_sc_mesh = plsc.VectorSubcoreMesh(core_axis_name="c", subcore_axis_name="s")
_sc_params = pltpu.CompilerParams(needs_layout_passes=False)


@functools.partial(
    pl.kernel,
    out_type=jax.ShapeDtypeStruct((_E,), jnp.float32),
    mesh=_sc_mesh,
    scratch_types=[
        pltpu.VMEM((_N,), jnp.float32),
        pltpu.VMEM((_N,), jnp.float32),
        pltpu.VMEM((_N,), jnp.float32),
        pltpu.VMEM((_EPW,), jnp.int32),
        pltpu.VMEM((_EPW,), jnp.int32),
        pltpu.VMEM((_EPW,), jnp.float32),
    ],
    compiler_params=_sc_params,
)
def _geom(px_h, py_h, pz_h, src_h, dst_h, dsq_h, px, py, pz, sidx, didx, dsq):
    c = lax.axis_index("c")
    s = lax.axis_index("s")
    w = s * _NC + c
    base = pl.multiple_of(w * _EPW, 8)
    pltpu.sync_copy(px_h, px)
    pltpu.sync_copy(py_h, py)
    pltpu.sync_copy(pz_h, pz)
    pltpu.sync_copy(src_h.at[pl.ds(base, _EPW)], sidx)
    pltpu.sync_copy(dst_h.at[pl.ds(base, _EPW)], didx)

    def body(i, carry):
        off = i * 16
        iv = sidx[pl.ds(off, 16)]
        jv = didx[pl.ds(off, 16)]
        dx = plsc.load_gather(px, [iv]) - plsc.load_gather(px, [jv])
        dy = plsc.load_gather(py, [iv]) - plsc.load_gather(py, [jv])
        dz = plsc.load_gather(pz, [iv]) - plsc.load_gather(pz, [jv])
        dsq[pl.ds(off, 16)] = dx * dx + dy * dy + dz * dz
        return carry

    lax.fori_loop(0, _EPW // 16, body, 0)
    pltpu.sync_copy(dsq, dsq_h.at[pl.ds(base, _EPW)])


@functools.partial(
    pl.kernel,
    out_type=jax.ShapeDtypeStruct((_NC, _NP, _D), jnp.float32),
    mesh=_sc_mesh,
    scratch_types=[
        pltpu.VMEM((_CH,), jnp.int32),
        pltpu.VMEM((_CH,), jnp.int32),
        pltpu.VMEM((_CH, _D), jnp.float32),
        pltpu.VMEM((_CH, _D), jnp.float32),
        pltpu.VMEM_SHARED((_NP, _D), jnp.float32),
        pltpu.SemaphoreType.DMA,
    ],
    compiler_params=_sc_params,
)
def _msg(m_h, w_h, src_h, dst_h, zeros_h, agg_h,
         sidx, didx, mrows, wrows, agg_sh, sem):
    c = lax.axis_index("c")
    s = lax.axis_index("s")
    w = s * _NC + c
    pltpu.sync_copy(zeros_h.at[pl.ds(s * _RPS, _RPS)],
                    agg_sh.at[pl.ds(s * _RPS, _RPS)])
    plsc.subcore_barrier()

    def body(j, carry):
        r = w * _CPW + j
        pltpu.sync_copy(src_h.at[r], sidx)
        pltpu.sync_copy(dst_h.at[r], didx)
        pltpu.async_copy(m_h.at[sidx], mrows, sem).wait()
        pltpu.sync_copy(w_h.at[pl.ds(r * _CH, _CH)], wrows)

        def mul(e, carry2):
            for cc in range(_D // 16):
                sl = pl.ds(cc * 16, 16)
                mrows[e, sl] = mrows[e, sl] * wrows[e, sl]
            return carry2

        lax.fori_loop(0, _CH, mul, 0)
        pltpu.sync_copy(mrows, agg_sh.at[didx], add=True)
        return carry

    lax.fori_loop(0, _CPW, body, 0)
    plsc.subcore_barrier()
    pltpu.sync_copy(agg_sh.at[pl.ds(s * _RPS, _RPS)],
                    agg_h.at[c, pl.ds(s * _RPS, _RPS)])


def _embed_body(z_ref, tab_ref, embW_ref, embb_ref, w0_ref, b0_ref,
                x_ref, m_ref):
    w2 = _mm(tab_ref[...], embW_ref[...])
    iot = lax.broadcasted_iota(jnp.int32, (_TN, 128), 1)
    oh = (iot == z_ref[...]).astype(jnp.float32)
    x = _mm(oh, w2) + embb_ref[...]
    x_ref[...] = x
    m_ref[...] = _mm(x, w0_ref[...]) + b0_ref[...]


def _embed(zp, tabp, embW, embb, w0, b0):
    return pl.pallas_call(
        _embed_body,
        grid=(_NP // _TN,),
        in_specs=[
            pl.BlockSpec((_TN, 1), lambda b: (b, 0)),
            pl.BlockSpec((128, _HID), lambda b: (0, 0)),
            pl.BlockSpec((_HID, _D), lambda b: (0, 0)),
            pl.BlockSpec((1, _D), lambda b: (0, 0)),
            pl.BlockSpec((_D, _D), lambda b: (0, 0)),
            pl.BlockSpec((1, _D), lambda b: (0, 0)),
        ],
        out_specs=[
            pl.BlockSpec((_TN, _D), lambda b: (b, 0)),
            pl.BlockSpec((_TN, _D), lambda b: (b, 0)),
        ],
        out_shape=[
            jax.ShapeDtypeStruct((_NP, _D), jnp.float32),
            jax.ShapeDtypeStruct((_NP, _D), jnp.float32),
        ],
    )(zp, tabp, embW, embb, w0, b0)


def _rbfw_body(dsq_ref, freq_ref, rbfW_ref, rbfb_ref, w4_ref):
    d = jnp.sqrt(dsq_ref[...] + 1e-12)
    x = jnp.maximum(d / _CUTOFF, 1e-6)
    x2 = x * x
    x4 = x2 * x2
    x5 = x4 * x
    x6 = x5 * x
    env = 1.0 / x + (-21.0) * x4 + 35.0 * x5 + (-15.0) * x6
    rbf = env * jnp.sin(freq_ref[...] * x)
    for blk in range(_NB):
        w4_ref[blk] = _mm(rbf, rbfW_ref[blk]) + rbfb_ref[blk]


def _rbfw(dsq2, freq2, rbfW, rbfb3):
    return pl.pallas_call(
        _rbfw_body,
        grid=(_E // _TE,),
        in_specs=[
            pl.BlockSpec((_TE, 1), lambda b: (b, 0)),
            pl.BlockSpec((1, _R), lambda b: (0, 0)),
            pl.BlockSpec((_NB, _R, _D), lambda b: (0, 0, 0)),
            pl.BlockSpec((_NB, 1, _D), lambda b: (0, 0, 0)),
        ],
        out_specs=pl.BlockSpec((_NB, _TE, _D), lambda b: (0, b, 0)),
        out_shape=jax.ShapeDtypeStruct((_NB, _E, _D), jnp.float32),
    )(dsq2, freq2, rbfW, rbfb3)


def _make_upd(has_next):
    def body(*refs):
        if has_next:
            (x_ref, agg_ref, woW_ref, wob_ref, f0W_ref, f0b_ref,
             f1W_ref, f1b_ref, wnW_ref, wnb_ref, xo_ref, mo_ref) = refs
        else:
            (x_ref, agg_ref, woW_ref, wob_ref, f0W_ref, f0b_ref,
             f1W_ref, f1b_ref, xo_ref) = refs
        a = agg_ref[0] + agg_ref[1]
        x = x_ref[...] + _mm(a, woW_ref[...]) + wob_ref[...]
        x = x + jax.nn.gelu(_mm(x, f0W_ref[...]) + f0b_ref[...])
        x = x + jax.nn.gelu(_mm(x, f1W_ref[...]) + f1b_ref[...])
        xo_ref[...] = x
        if has_next:
            mo_ref[...] = _mm(x, wnW_ref[...]) + wnb_ref[...]

    full_w = pl.BlockSpec((_D, _D), lambda b: (0, 0))
    full_b = pl.BlockSpec((1, _D), lambda b: (0, 0))
    tile = pl.BlockSpec((_TN, _D), lambda b: (b, 0))
    in_specs = [
        tile,
        pl.BlockSpec((_NC, _TN, _D), lambda b: (0, b, 0)),
        full_w, full_b, full_w, full_b, full_w, full_b,
    ]
    out_shape = [jax.ShapeDtypeStruct((_NP, _D), jnp.float32)]
    out_specs = [tile]
    if has_next:
        in_specs += [full_w, full_b]
        out_shape.append(jax.ShapeDtypeStruct((_NP, _D), jnp.float32))
        out_specs.append(tile)

    def run(*args):
        return pl.pallas_call(
            body,
            grid=(_NP // _TN,),
            in_specs=in_specs,
            out_specs=out_specs,
            out_shape=out_shape,
        )(*args)

    return run


_upd_next = _make_upd(True)
_upd_last = _make_upd(False)


def kernel(z, pos, batch, ptr, edge_index, emb_table, emb_W, emb_b, freq,
           msg_in_W, msg_in_b, rbf_W, rbf_b, msg_out_W, msg_out_b, fc_W, fc_b):
    src = edge_index[0].astype(jnp.int32)
    dst = edge_index[1].astype(jnp.int32)
    posf = pos.astype(jnp.float32)

    dsq = _geom(posf[:, 0], posf[:, 1], posf[:, 2], src, dst)

    w4 = _rbfw(dsq.reshape(_E, 1), freq.reshape(1, _R), rbf_W,
               rbf_b.reshape(_NB, 1, _D))

    zp = jnp.zeros((_NP, 1), jnp.int32).at[:_N].set(z.astype(jnp.int32))
    tabp = jnp.zeros((128, _HID), jnp.float32).at[:_NTYPES].set(emb_table)
    x, m = _embed(zp, tabp, emb_W, emb_b.reshape(1, _D),
                  msg_in_W[0], msg_in_b[0].reshape(1, _D))

    zeros_np = jnp.zeros((_NP, _D), jnp.float32)
    src2 = src.reshape(_NCHUNK, _CH)
    dst2 = dst.reshape(_NCHUNK, _CH)

    for blk in range(_NB):
        aggp = _msg(m, w4[blk], src2, dst2, zeros_np)
        wo = msg_out_W[blk]
        wob = msg_out_b[blk].reshape(1, _D)
        f0W = fc_W[blk, 0]
        f0b = fc_b[blk, 0].reshape(1, _D)
        f1W = fc_W[blk, 1]
        f1b = fc_b[blk, 1].reshape(1, _D)
        if blk < _NB - 1:
            x, m = _upd_next(x, aggp, wo, wob, f0W, f0b, f1W, f1b,
                             msg_in_W[blk + 1],
                             msg_in_b[blk + 1].reshape(1, _D))
        else:
            (x,) = _upd_last(x, aggp, wo, wob, f0W, f0b, f1W, f1b)

    return x[:_N]
